# Initial kernel scaffold; baseline (speedup 1.0000x reference)
import jax
import jax.numpy as jnp
from jax.experimental import pallas as pl


def kernel(fea, adj_index, adj_weight, adj_knn_index, adj_knn_weight, W_in, W_in_self, b_in, W_out, W_out_self, b_out, scores0, bias0, Dk0, Dbias0):
    raise NotImplementedError("write your pallas kernel here")


# speedup vs baseline: 3.3042x; 3.3042x over previous
import functools

import jax
import jax.numpy as jnp
from jax import lax
from jax.experimental import pallas as pl
from jax.experimental.pallas import tpu as pltpu
from jax.experimental.pallas import tpu_sc as plsc

_GAMMA = 0.1
_B = 128
_NW = 32
_ROW_BLK = 1024
_N_PAD = 10240


def _lane_bcast(v16, lane):
    idx = (jnp.zeros((16,), jnp.int32) + lane)[:, None]
    return lax.gather(
        v16, idx,
        lax.GatherDimensionNumbers(
            offset_dims=(), collapsed_slice_dims=(0,), start_index_map=(0,)),
        slice_sizes=(1,),
        mode=lax.GatherScatterMode.PROMISE_IN_BOUNDS)


def _make_sc_spmm(n, h, blocks_adj, blocks_knn):
    rpt = n // 16
    mesh = plsc.VectorSubcoreMesh(
        core_axis_name="c", subcore_axis_name="s", num_cores=2,
        num_subcores=16)

    @functools.partial(
        pl.kernel,
        out_type=jax.ShapeDtypeStruct((2, n, h), jnp.float32),
        mesh=mesh,
        scratch_types=[
            pltpu.VMEM((n,), jnp.float32),
            pltpu.VMEM((_B,), jnp.int32),
            pltpu.VMEM((_B,), jnp.int32),
            pltpu.VMEM((_B,), jnp.float32),
            pltpu.VMEM((_B,), jnp.float32),
            pltpu.VMEM((_B, h), jnp.float32),
            pltpu.VMEM((16, h), jnp.float32),
            pltpu.VMEM_SHARED((n, h), jnp.float32),
            pltpu.SemaphoreType.DMA,
        ],
        compiler_params=pltpu.CompilerParams(
            needs_layout_passes=False, use_tc_tiling_on_sc=False),
    )
    def spmm_kernel(s_hbm, tab_hbm, asrc, adst, aw, ksrc, kdst, kw,
                    init0, out_hbm,
                    s_v, src_v, dst_v, w_v, ws_v, rows_v, z_v, acc, sem):
        c = lax.axis_index("c")
        s = lax.axis_index("s")
        wid = c * 16 + s
        r0 = s * rpt

        @pl.when(c == 0)
        def _():
            pltpu.sync_copy(init0.at[pl.ds(r0, rpt)], acc.at[pl.ds(r0, rpt)])

        @pl.when(c != 0)
        def _():
            zero = jnp.zeros((16,), jnp.float32)
            for r in range(16):
                for k in range(h // 16):
                    z_v[r, pl.ds(k * 16, 16)] = zero

            def zblk(j, carry):
                pltpu.sync_copy(z_v, acc.at[pl.ds(r0 + j * 16, 16)])
                return carry

            lax.fori_loop(0, rpt // 16, zblk, 0)

        pltpu.sync_copy(s_hbm, s_v)
        plsc.subcore_barrier()

        def run_list(src_hbm, dst_hbm, w_hbm, nblocks, knn):
            base_w = wid * (nblocks * _B)

            def blk(i, carry):
                base = base_w + i * _B
                pltpu.sync_copy(src_hbm.at[pl.ds(base, _B)], src_v)
                pltpu.sync_copy(dst_hbm.at[pl.ds(base, _B)], dst_v)
                pltpu.sync_copy(w_hbm.at[pl.ds(base, _B)], w_v)
                pltpu.async_copy(tab_hbm.at[src_v], rows_v, sem).wait()
                for g in range(_B // 16):
                    sl = pl.ds(g * 16, 16)
                    sg = plsc.load_gather(s_v, [dst_v[sl]])
                    gate = (1.0 - sg) if knn else sg
                    ws_v[sl] = w_v[sl] * gate

                def scale(e, carry2):
                    grp = pl.multiple_of((e // 16) * 16, 16)
                    w16 = ws_v[pl.ds(grp, 16)]
                    wb = _lane_bcast(w16, e - grp)
                    for k in range(h // 16):
                        cs = pl.ds(k * 16, 16)
                        rows_v[e, cs] = rows_v[e, cs] * wb
                    return carry2

                lax.fori_loop(0, _B, scale, 0)
                pltpu.sync_copy(rows_v, acc.at[dst_v], add=True)
                return carry

            lax.fori_loop(0, nblocks, blk, 0)

        run_list(asrc, adst, aw, blocks_adj, False)
        run_list(ksrc, kdst, kw, blocks_knn, True)
        plsc.subcore_barrier()
        pltpu.sync_copy(acc.at[pl.ds(r0, rpt)],
                        out_hbm.at[c, pl.ds(r0, rpt)])

    return spmm_kernel


def _sigmoid(z):
    return 1.0 / (1.0 + jnp.exp(-z))


def _tc_layer1(fea, W_in, W_in_self, b_in, scores0, Dk0, bias0, Dbias0):
    n, f = fea.shape
    hh = W_in.shape[1]
    grid = (n // _ROW_BLK,)

    def body(f_ref, win_ref, wins_ref, bin_ref, sc_ref, dk_ref, b0_ref,
             db_ref, s_out, d_out, sig_out):
        x = f_ref[...]
        S = jnp.dot(x, win_ref[...], preferred_element_type=jnp.float32)
        sid = _sigmoid(
            jnp.dot(x, sc_ref[...], preferred_element_type=jnp.float32)
            + b0_ref[...])
        dk = jnp.dot(x, dk_ref[...], preferred_element_type=jnp.float32) \
            + db_ref[...]
        self_t = jnp.dot(x, wins_ref[...],
                         preferred_element_type=jnp.float32) + bin_ref[...]
        D = self_t + _GAMMA * dk * (S + self_t)
        s_out[...] = S
        d_out[...] = D
        sig_out[...] = sid

    return pl.pallas_call(
        body,
        grid=grid,
        in_specs=[
            pl.BlockSpec((_ROW_BLK, f), lambda i: (i, 0)),
            pl.BlockSpec((f, hh), lambda i: (0, 0)),
            pl.BlockSpec((f, hh), lambda i: (0, 0)),
            pl.BlockSpec((hh,), lambda i: (0,)),
            pl.BlockSpec((f, 1), lambda i: (0, 0)),
            pl.BlockSpec((f, 1), lambda i: (0, 0)),
            pl.BlockSpec((1,), lambda i: (0,)),
            pl.BlockSpec((1,), lambda i: (0,)),
        ],
        out_specs=[
            pl.BlockSpec((_ROW_BLK, hh), lambda i: (i, 0)),
            pl.BlockSpec((_ROW_BLK, hh), lambda i: (i, 0)),
            pl.BlockSpec((_ROW_BLK, 1), lambda i: (i, 0)),
        ],
        out_shape=[
            jax.ShapeDtypeStruct((n, hh), jnp.float32),
            jax.ShapeDtypeStruct((n, hh), jnp.float32),
            jax.ShapeDtypeStruct((n, 1), jnp.float32),
        ],
    )(fea, W_in, W_in_self, b_in, scores0, Dk0, bias0, Dbias0)


def _tc_layer2(parts, W_out, W_out_self, b_out, scores0, Dk0, bias0, Dbias0):
    _, n, hh = parts.shape
    cc = W_out.shape[1]
    grid = (n // _ROW_BLK,)

    def body(p_ref, wo_ref, wos_ref, bo_ref, sc_ref, dk_ref, b0_ref,
             db_ref, s_out, d_out, sig_out):
        x = p_ref[0] + p_ref[1]
        S = jnp.dot(x, wo_ref[...], preferred_element_type=jnp.float32)
        sid = _sigmoid(
            jnp.dot(x, sc_ref[...], preferred_element_type=jnp.float32)
            + b0_ref[...])
        dk = jnp.dot(x, dk_ref[...], preferred_element_type=jnp.float32) \
            + db_ref[...]
        self_t = jnp.dot(x, wos_ref[...],
                         preferred_element_type=jnp.float32) + bo_ref[...]
        D = self_t + _GAMMA * dk * (S + self_t)
        s_out[...] = S
        d_out[...] = D
        sig_out[...] = sid

    return pl.pallas_call(
        body,
        grid=grid,
        in_specs=[
            pl.BlockSpec((2, _ROW_BLK, hh), lambda i: (0, i, 0)),
            pl.BlockSpec((hh, cc), lambda i: (0, 0)),
            pl.BlockSpec((hh, cc), lambda i: (0, 0)),
            pl.BlockSpec((cc,), lambda i: (0,)),
            pl.BlockSpec((hh, 1), lambda i: (0, 0)),
            pl.BlockSpec((hh, 1), lambda i: (0, 0)),
            pl.BlockSpec((1,), lambda i: (0,)),
            pl.BlockSpec((1,), lambda i: (0,)),
        ],
        out_specs=[
            pl.BlockSpec((_ROW_BLK, cc), lambda i: (i, 0)),
            pl.BlockSpec((_ROW_BLK, cc), lambda i: (i, 0)),
            pl.BlockSpec((_ROW_BLK, 1), lambda i: (i, 0)),
        ],
        out_shape=[
            jax.ShapeDtypeStruct((n, cc), jnp.float32),
            jax.ShapeDtypeStruct((n, cc), jnp.float32),
            jax.ShapeDtypeStruct((n, 1), jnp.float32),
        ],
    )(parts, W_out, W_out_self, b_out, scores0, Dk0, bias0, Dbias0)


def _tc_final(parts):
    _, n, cc = parts.shape
    grid = (n // _ROW_BLK,)

    def body(p_ref, o_ref):
        z = p_ref[0] + p_ref[1]
        m = jnp.max(z, axis=1, keepdims=True)
        zm = z - m
        o_ref[...] = zm - jnp.log(jnp.sum(jnp.exp(zm), axis=1, keepdims=True))

    return pl.pallas_call(
        body,
        grid=grid,
        in_specs=[pl.BlockSpec((2, _ROW_BLK, cc), lambda i: (0, i, 0))],
        out_specs=pl.BlockSpec((_ROW_BLK, cc), lambda i: (i, 0)),
        out_shape=jax.ShapeDtypeStruct((n, cc), jnp.float32),
    )(parts)


def _pad_edges(index, weight):
    e = weight.shape[0]
    unit = _NW * _B
    epad = ((e + unit - 1) // unit) * unit
    pad = epad - e
    src = jnp.concatenate([index[0], jnp.zeros((pad,), jnp.int32)])
    dst = jnp.concatenate([index[1], jnp.zeros((pad,), jnp.int32)])
    w = jnp.concatenate([weight, jnp.zeros((pad,), jnp.float32)])
    return src, dst, w, epad // _NW // _B


def kernel(fea, adj_index, adj_weight, adj_knn_index, adj_knn_weight,
           W_in, W_in_self, b_in, W_out, W_out_self, b_out,
           scores0, bias0, Dk0, Dbias0):
    n_real = fea.shape[0]
    n = _N_PAD
    fea = jnp.pad(fea, ((0, n - n_real), (0, 0)))
    hh = W_in.shape[1]
    cc = W_out.shape[1]

    asrc, adst, aw, blocks_adj = _pad_edges(adj_index, adj_weight)
    ksrc, kdst, kw, blocks_knn = _pad_edges(adj_knn_index, adj_knn_weight)

    S1, D1, sig1 = _tc_layer1(fea, W_in, W_in_self, b_in, scores0, Dk0,
                              bias0, Dbias0)
    sc1 = _make_sc_spmm(n, hh, blocks_adj, blocks_knn)
    parts1 = sc1(sig1.reshape(n), S1, asrc, adst, aw, ksrc, kdst, kw, D1)

    S2, D2, sig2 = _tc_layer2(parts1, W_out, W_out_self, b_out, scores0,
                              Dk0, bias0, Dbias0)
    sc2 = _make_sc_spmm(n, cc, blocks_adj, blocks_knn)
    parts2 = sc2(sig2.reshape(n), S2, asrc, adst, aw, ksrc, kdst, kw, D2)

    return _tc_final(parts2)[:n_real]


# speedup vs baseline: 5.9988x; 1.8155x over previous
import functools

import jax
import jax.numpy as jnp
from jax import lax
from jax.experimental import pallas as pl
from jax.experimental.pallas import tpu as pltpu
from jax.experimental.pallas import tpu_sc as plsc

_GAMMA = 0.1
_B = 128
_NW = 32
_ROW_BLK = 1024
_N_PAD = 10240


def _lane_bcast(v16, lane):
    idx = jnp.full((16, 1), lane, jnp.int32)
    return lax.gather(
        v16, idx,
        lax.GatherDimensionNumbers(
            offset_dims=(), collapsed_slice_dims=(0,), start_index_map=(0,)),
        slice_sizes=(1,),
        mode=lax.GatherScatterMode.PROMISE_IN_BOUNDS)


_CH = 4


def _make_sc_spmm(n, h, blocks_adj, nb):
    rpt = n // 16
    assert nb % _CH == 0
    nchunks = nb // _CH
    ngrp = _B // 16
    mesh = plsc.VectorSubcoreMesh(
        core_axis_name="c", subcore_axis_name="s", num_cores=2,
        num_subcores=16)

    @functools.partial(
        pl.kernel,
        out_type=jax.ShapeDtypeStruct((2, n, h), jnp.float32),
        mesh=mesh,
        scratch_types=[
            pltpu.VMEM((n,), jnp.float32),
            pltpu.VMEM((2, _CH, 3, _B), jnp.int32),
            pltpu.VMEM((_B, h), jnp.float32),
            pltpu.VMEM((_B, h), jnp.float32),
            pltpu.VMEM((8, h), jnp.float32),
            pltpu.VMEM_SHARED((n, h), jnp.float32),
            pltpu.SemaphoreType.DMA,
            pltpu.SemaphoreType.DMA,
            pltpu.SemaphoreType.DMA,
            pltpu.SemaphoreType.DMA,
            pltpu.SemaphoreType.DMA,
        ],
        compiler_params=pltpu.CompilerParams(
            needs_layout_passes=False, use_tc_tiling_on_sc=False),
    )
    def spmm_kernel(s_hbm, tab_hbm, comb_hbm, init0, out_hbm,
                    s_v, comb_v, rows0, rows1, z_v, acc,
                    semg0, semg1, sems0, sems1, semc):
        c = lax.axis_index("c")
        s = lax.axis_index("s")
        wid = c * 16 + s
        r0 = s * rpt

        @pl.when(c == 0)
        def _():
            pltpu.sync_copy(init0.at[pl.ds(r0, rpt)], acc.at[pl.ds(r0, rpt)])

        @pl.when(c != 0)
        def _():
            zero = jnp.zeros((16,), jnp.float32)
            for r in range(8):
                for k in range(h // 16):
                    z_v[r, pl.ds(k * 16, 16)] = zero

            def zblk(j, carry):
                pltpu.sync_copy(z_v, acc.at[pl.ds(r0 + j * 8, 8)])
                return carry

            lax.fori_loop(0, rpt // 8, zblk, 0)

        pltpu.sync_copy(s_hbm, s_v)
        pltpu.sync_copy(comb_hbm.at[wid, pl.ds(0, _CH)], comb_v.at[0])
        plsc.subcore_barrier()

        def stage_start(q):
            pltpu.async_copy(comb_hbm.at[wid, pl.ds(q * _CH, _CH)],
                             comb_v.at[q % 2], semc)

        def stage_wait(q):
            pltpu.make_async_copy(comb_hbm.at[wid, pl.ds(q * _CH, _CH)],
                                  comb_v.at[q % 2], semc).wait()

        def gather_start(qp, b, rows, semg):
            pltpu.async_copy(tab_hbm.at[comb_v.at[qp, b, 0]], rows, semg)

        def gather_wait(qp, b, rows, semg):
            pltpu.make_async_copy(tab_hbm.at[comb_v.at[qp, b, 0]], rows,
                                  semg).wait()

        def scatter_start(qp, b, rows, sems):
            pltpu.async_copy(rows, acc.at[comb_v.at[qp, b, 1]], sems,
                             add=True)

        def scatter_wait(qp, b, rows, sems):
            pltpu.make_async_copy(rows, acc.at[comb_v.at[qp, b, 1]],
                                  sems).wait()

        def scale_half(i, qp, b, rows, g_lo, g_hi):
            def grp(g, carry):
                gs = pl.ds(g * 16, 16)
                dst16 = comb_v[qp, b, 1, gs]
                w16 = plsc.bitcast(comb_v[qp, b, 2, gs], jnp.float32)
                sg = plsc.load_gather(s_v, [dst16])
                gate = jnp.where(i >= blocks_adj, 1.0 - sg, sg)
                ws16 = w16 * gate
                for lane in range(16):
                    wb = _lane_bcast(ws16, lane)
                    e = g * 16 + lane
                    for k in range(h // 16):
                        cs = pl.ds(k * 16, 16)
                        rows[e, cs] = rows[e, cs] * wb
                return carry

            lax.fori_loop(g_lo, g_hi, grp, 0)

        bufs = {0: (rows0, rows1, semg0, semg1, sems0, sems1),
                1: (rows1, rows0, semg1, semg0, sems1, sems0)}
        gather_start(0, 0, rows0, semg0)

        def body(q, carry):
            qp = q % 2

            for b in range(_CH):
                i = q * _CH + b
                rows, ro, semg, semg_o, sems, sems_o = bufs[b % 2]
                gather_wait(qp, b, rows, semg)
                scale_half(i, qp, b, rows, 0, ngrp // 2)

                if b > 0:
                    scatter_wait(qp, b - 1, ro, sems_o)
                else:
                    @pl.when(q >= 1)
                    def _():
                        scatter_wait(1 - qp, _CH - 1, ro, sems_o)

                    @pl.when(q + 1 < nchunks)
                    def _():
                        stage_start(q + 1)

                if b < _CH - 1:
                    gather_start(qp, b + 1, ro, semg_o)
                else:
                    @pl.when(q + 1 < nchunks)
                    def _():
                        stage_wait(q + 1)
                        gather_start(1 - qp, 0, ro, semg_o)

                scale_half(i, qp, b, rows, ngrp // 2, ngrp)
                scatter_start(qp, b, rows, sems)
            return carry

        lax.fori_loop(0, nchunks, body, 0)
        rows_l, _, _, _, sems_l, _ = bufs[(_CH - 1) % 2]
        scatter_wait((nchunks - 1) % 2, _CH - 1, rows_l, sems_l)
        plsc.subcore_barrier()
        pltpu.sync_copy(acc.at[pl.ds(r0, rpt)],
                        out_hbm.at[c, pl.ds(r0, rpt)])

    return spmm_kernel


def _sigmoid(z):
    return 1.0 / (1.0 + jnp.exp(-z))


def _tc_layer1(fea, W_in, W_in_self, b_in, scores0, Dk0, bias0, Dbias0):
    n, f = fea.shape
    hh = W_in.shape[1]
    grid = (n // _ROW_BLK,)

    def body(f_ref, win_ref, wins_ref, bin_ref, sc_ref, dk_ref, b0_ref,
             db_ref, s_out, d_out, sig_out):
        x = f_ref[...]
        S = jnp.dot(x, win_ref[...], preferred_element_type=jnp.float32)
        sid = _sigmoid(
            jnp.dot(x, sc_ref[...], preferred_element_type=jnp.float32)
            + b0_ref[...])
        dk = jnp.dot(x, dk_ref[...], preferred_element_type=jnp.float32) \
            + db_ref[...]
        self_t = jnp.dot(x, wins_ref[...],
                         preferred_element_type=jnp.float32) + bin_ref[...]
        D = self_t + _GAMMA * dk * (S + self_t)
        s_out[...] = S
        d_out[...] = D
        sig_out[...] = sid

    return pl.pallas_call(
        body,
        grid=grid,
        in_specs=[
            pl.BlockSpec((_ROW_BLK, f), lambda i: (i, 0)),
            pl.BlockSpec((f, hh), lambda i: (0, 0)),
            pl.BlockSpec((f, hh), lambda i: (0, 0)),
            pl.BlockSpec((hh,), lambda i: (0,)),
            pl.BlockSpec((f, 1), lambda i: (0, 0)),
            pl.BlockSpec((f, 1), lambda i: (0, 0)),
            pl.BlockSpec((1,), lambda i: (0,)),
            pl.BlockSpec((1,), lambda i: (0,)),
        ],
        out_specs=[
            pl.BlockSpec((_ROW_BLK, hh), lambda i: (i, 0)),
            pl.BlockSpec((_ROW_BLK, hh), lambda i: (i, 0)),
            pl.BlockSpec((_ROW_BLK, 1), lambda i: (i, 0)),
        ],
        out_shape=[
            jax.ShapeDtypeStruct((n, hh), jnp.float32),
            jax.ShapeDtypeStruct((n, hh), jnp.float32),
            jax.ShapeDtypeStruct((n, 1), jnp.float32),
        ],
    )(fea, W_in, W_in_self, b_in, scores0, Dk0, bias0, Dbias0)


def _tc_layer2(parts, W_out, W_out_self, b_out, scores0, Dk0, bias0, Dbias0):
    _, n, hh = parts.shape
    cc = W_out.shape[1]
    grid = (n // _ROW_BLK,)

    def body(p_ref, wo_ref, wos_ref, bo_ref, sc_ref, dk_ref, b0_ref,
             db_ref, s_out, d_out, sig_out):
        x = p_ref[0] + p_ref[1]
        S = jnp.dot(x, wo_ref[...], preferred_element_type=jnp.float32)
        sid = _sigmoid(
            jnp.dot(x, sc_ref[...], preferred_element_type=jnp.float32)
            + b0_ref[...])
        dk = jnp.dot(x, dk_ref[...], preferred_element_type=jnp.float32) \
            + db_ref[...]
        self_t = jnp.dot(x, wos_ref[...],
                         preferred_element_type=jnp.float32) + bo_ref[...]
        D = self_t + _GAMMA * dk * (S + self_t)
        s_out[...] = S
        d_out[...] = D
        sig_out[...] = sid

    return pl.pallas_call(
        body,
        grid=grid,
        in_specs=[
            pl.BlockSpec((2, _ROW_BLK, hh), lambda i: (0, i, 0)),
            pl.BlockSpec((hh, cc), lambda i: (0, 0)),
            pl.BlockSpec((hh, cc), lambda i: (0, 0)),
            pl.BlockSpec((cc,), lambda i: (0,)),
            pl.BlockSpec((hh, 1), lambda i: (0, 0)),
            pl.BlockSpec((hh, 1), lambda i: (0, 0)),
            pl.BlockSpec((1,), lambda i: (0,)),
            pl.BlockSpec((1,), lambda i: (0,)),
        ],
        out_specs=[
            pl.BlockSpec((_ROW_BLK, cc), lambda i: (i, 0)),
            pl.BlockSpec((_ROW_BLK, cc), lambda i: (i, 0)),
            pl.BlockSpec((_ROW_BLK, 1), lambda i: (i, 0)),
        ],
        out_shape=[
            jax.ShapeDtypeStruct((n, cc), jnp.float32),
            jax.ShapeDtypeStruct((n, cc), jnp.float32),
            jax.ShapeDtypeStruct((n, 1), jnp.float32),
        ],
    )(parts, W_out, W_out_self, b_out, scores0, Dk0, bias0, Dbias0)


def _tc_final(parts):
    _, n, cc = parts.shape
    grid = (n // _ROW_BLK,)

    def body(p_ref, o_ref):
        z = p_ref[0] + p_ref[1]
        m = jnp.max(z, axis=1, keepdims=True)
        zm = z - m
        o_ref[...] = zm - jnp.log(jnp.sum(jnp.exp(zm), axis=1, keepdims=True))

    return pl.pallas_call(
        body,
        grid=grid,
        in_specs=[pl.BlockSpec((2, _ROW_BLK, cc), lambda i: (0, i, 0))],
        out_specs=pl.BlockSpec((_ROW_BLK, cc), lambda i: (i, 0)),
        out_shape=jax.ShapeDtypeStruct((n, cc), jnp.float32),
    )(parts)


def _pack_edges(index, weight):
    e = weight.shape[0]
    unit = _NW * _B
    epad = ((e + unit - 1) // unit) * unit
    pad = epad - e
    src = jnp.concatenate([index[0], jnp.zeros((pad,), jnp.int32)])
    dst = jnp.concatenate([index[1], jnp.zeros((pad,), jnp.int32)])
    w = jnp.concatenate([weight, jnp.zeros((pad,), jnp.float32)])
    wi = lax.bitcast_convert_type(w, jnp.int32)
    nb = epad // _NW // _B
    comb = jnp.stack([x.reshape(_NW, nb, _B) for x in (src, dst, wi)],
                     axis=2)
    return comb, nb


def kernel(fea, adj_index, adj_weight, adj_knn_index, adj_knn_weight,
           W_in, W_in_self, b_in, W_out, W_out_self, b_out,
           scores0, bias0, Dk0, Dbias0):
    n_real = fea.shape[0]
    n = _N_PAD
    fea = jnp.pad(fea, ((0, n - n_real), (0, 0)))
    hh = W_in.shape[1]
    cc = W_out.shape[1]

    comb_adj, blocks_adj = _pack_edges(adj_index, adj_weight)
    comb_knn, blocks_knn = _pack_edges(adj_knn_index, adj_knn_weight)
    nb = blocks_adj + blocks_knn
    nb_pad = ((nb + _CH - 1) // _CH) * _CH
    comb = jnp.concatenate(
        [comb_adj, comb_knn] +
        ([jnp.zeros((_NW, nb_pad - nb, 3, _B), jnp.int32)]
         if nb_pad > nb else []), axis=1)
    nb = nb_pad

    S1, D1, sig1 = _tc_layer1(fea, W_in, W_in_self, b_in, scores0, Dk0,
                              bias0, Dbias0)
    sc1 = _make_sc_spmm(n, hh, blocks_adj, nb)
    parts1 = sc1(sig1.reshape(n), S1, comb, D1)

    S2, D2, sig2 = _tc_layer2(parts1, W_out, W_out_self, b_out, scores0,
                              Dk0, bias0, Dbias0)
    sc2 = _make_sc_spmm(n, cc, blocks_adj, nb)
    parts2 = sc2(sig2.reshape(n), S2, comb, D2)

    return _tc_final(parts2)[:n_real]


# speedup vs baseline: 6.4527x; 1.0757x over previous
import functools

import jax
import jax.numpy as jnp
from jax import lax
from jax.experimental import pallas as pl
from jax.experimental.pallas import tpu as pltpu
from jax.experimental.pallas import tpu_sc as plsc

_GAMMA = 0.1
_B = 64
_NW = 32
_ROW_BLK = 1024
_N_PAD = 10240


def _lane_bcast(v16, lane):
    idx = jnp.full((16, 1), lane, jnp.int32)
    return lax.gather(
        v16, idx,
        lax.GatherDimensionNumbers(
            offset_dims=(), collapsed_slice_dims=(0,), start_index_map=(0,)),
        slice_sizes=(1,),
        mode=lax.GatherScatterMode.PROMISE_IN_BOUNDS)


_CH = 4


def _make_sc_spmm(n, h, blocks_adj, nb):
    rpt = n // 16
    assert nb % _CH == 0
    nchunks = nb // _CH
    ngrp = _B // 16
    mesh = plsc.VectorSubcoreMesh(
        core_axis_name="c", subcore_axis_name="s", num_cores=2,
        num_subcores=16)

    @functools.partial(
        pl.kernel,
        out_type=jax.ShapeDtypeStruct((2, n, h), jnp.float32),
        mesh=mesh,
        scratch_types=[
            pltpu.VMEM((n,), jnp.float32),
            pltpu.VMEM((2, _CH, 3, _B), jnp.int32),
            pltpu.VMEM((_B, h), jnp.float32),
            pltpu.VMEM((_B, h), jnp.float32),
            pltpu.VMEM((_B, h), jnp.float32),
            pltpu.VMEM((_B, h), jnp.float32),
            pltpu.VMEM((8, h), jnp.float32),
            pltpu.VMEM_SHARED((n, h), jnp.float32),
            pltpu.SemaphoreType.DMA,
            pltpu.SemaphoreType.DMA,
            pltpu.SemaphoreType.DMA,
            pltpu.SemaphoreType.DMA,
            pltpu.SemaphoreType.DMA,
            pltpu.SemaphoreType.DMA,
            pltpu.SemaphoreType.DMA,
            pltpu.SemaphoreType.DMA,
            pltpu.SemaphoreType.DMA,
        ],
        compiler_params=pltpu.CompilerParams(
            needs_layout_passes=False, use_tc_tiling_on_sc=False),
    )
    def spmm_kernel(s_hbm, tab_hbm, comb_hbm, init0, out_hbm,
                    s_v, comb_v, rows0, rows1, rows2, rows3, z_v, acc,
                    semg0, semg1, semg2, semg3,
                    sems0, sems1, sems2, sems3, semc):
        c = lax.axis_index("c")
        s = lax.axis_index("s")
        wid = c * 16 + s
        r0 = s * rpt

        @pl.when(c == 0)
        def _():
            pltpu.sync_copy(init0.at[pl.ds(r0, rpt)], acc.at[pl.ds(r0, rpt)])

        @pl.when(c != 0)
        def _():
            zero = jnp.zeros((16,), jnp.float32)
            for r in range(8):
                for k in range(h // 16):
                    z_v[r, pl.ds(k * 16, 16)] = zero

            def zblk(j, carry):
                pltpu.sync_copy(z_v, acc.at[pl.ds(r0 + j * 8, 8)])
                return carry

            lax.fori_loop(0, rpt // 8, zblk, 0)

        pltpu.sync_copy(s_hbm, s_v)
        pltpu.sync_copy(comb_hbm.at[wid, pl.ds(0, _CH)], comb_v.at[0])
        plsc.subcore_barrier()

        def stage_start(q):
            pltpu.async_copy(comb_hbm.at[wid, pl.ds(q * _CH, _CH)],
                             comb_v.at[q % 2], semc)

        def stage_wait(q):
            pltpu.make_async_copy(comb_hbm.at[wid, pl.ds(q * _CH, _CH)],
                                  comb_v.at[q % 2], semc).wait()

        def gather_start(qp, b, rows, semg):
            pltpu.async_copy(tab_hbm.at[comb_v.at[qp, b, 0]], rows, semg)

        def gather_wait(qp, b, rows, semg):
            pltpu.make_async_copy(tab_hbm.at[comb_v.at[qp, b, 0]], rows,
                                  semg).wait()

        def scatter_start(qp, b, rows, sems):
            pltpu.async_copy(rows, acc.at[comb_v.at[qp, b, 1]], sems,
                             add=True)

        def scatter_wait(qp, b, rows, sems):
            pltpu.make_async_copy(rows, acc.at[comb_v.at[qp, b, 1]],
                                  sems).wait()

        def scale(i, qp, b, rows):
            def grp(g, carry):
                gs = pl.ds(g * 16, 16)
                dst16 = comb_v[qp, b, 1, gs]
                w16 = plsc.bitcast(comb_v[qp, b, 2, gs], jnp.float32)
                sg = plsc.load_gather(s_v, [dst16])
                gate = jnp.where(i >= blocks_adj, 1.0 - sg, sg)
                ws16 = w16 * gate
                for lane in range(16):
                    wb = _lane_bcast(ws16, lane)
                    e = g * 16 + lane
                    for k in range(h // 16):
                        cs = pl.ds(k * 16, 16)
                        rows[e, cs] = rows[e, cs] * wb
                return carry

            lax.fori_loop(0, ngrp, grp, 0)

        bufs = [(rows0, semg0, sems0), (rows1, semg1, sems1),
                (rows2, semg2, sems2), (rows3, semg3, sems3)]
        gather_start(0, 0, rows0, semg0)

        def body(q, carry):
            qp = q % 2

            for b in range(_CH):
                i = q * _CH + b
                rows, semg, sems = bufs[b]
                rn, semg_n, sems_n = bufs[(b + 1) % _CH]

                if b == _CH - 1:
                    scatter_wait(qp, 0, rn, sems_n)
                else:
                    @pl.when(q >= 1)
                    def _():
                        scatter_wait(1 - qp, b + 1, rn, sems_n)

                if b == 2:
                    @pl.when(q + 1 < nchunks)
                    def _():
                        stage_start(q + 1)

                if b < _CH - 1:
                    gather_start(qp, b + 1, rn, semg_n)
                else:
                    @pl.when(q + 1 < nchunks)
                    def _():
                        stage_wait(q + 1)
                        gather_start(1 - qp, 0, rn, semg_n)

                gather_wait(qp, b, rows, semg)
                scale(i, qp, b, rows)
                scatter_start(qp, b, rows, sems)
            return carry

        lax.fori_loop(0, nchunks, body, 0)
        lq = (nchunks - 1) % 2
        for b in range(1, _CH):
            rows_l, _, sems_l = bufs[b]
            scatter_wait(lq, b, rows_l, sems_l)
        plsc.subcore_barrier()
        pltpu.sync_copy(acc.at[pl.ds(r0, rpt)],
                        out_hbm.at[c, pl.ds(r0, rpt)])

    return spmm_kernel


def _sigmoid(z):
    return 1.0 / (1.0 + jnp.exp(-z))


def _tc_layer1(fea, W_in, W_in_self, b_in, scores0, Dk0, bias0, Dbias0):
    n, f = fea.shape
    hh = W_in.shape[1]
    grid = (n // _ROW_BLK,)

    def body(f_ref, win_ref, wins_ref, bin_ref, sc_ref, dk_ref, b0_ref,
             db_ref, s_out, d_out, sig_out):
        x = f_ref[...]
        S = jnp.dot(x, win_ref[...], preferred_element_type=jnp.float32)
        sid = _sigmoid(
            jnp.dot(x, sc_ref[...], preferred_element_type=jnp.float32)
            + b0_ref[...])
        dk = jnp.dot(x, dk_ref[...], preferred_element_type=jnp.float32) \
            + db_ref[...]
        self_t = jnp.dot(x, wins_ref[...],
                         preferred_element_type=jnp.float32) + bin_ref[...]
        D = self_t + _GAMMA * dk * (S + self_t)
        s_out[...] = S
        d_out[...] = D
        sig_out[...] = sid

    return pl.pallas_call(
        body,
        grid=grid,
        in_specs=[
            pl.BlockSpec((_ROW_BLK, f), lambda i: (i, 0)),
            pl.BlockSpec((f, hh), lambda i: (0, 0)),
            pl.BlockSpec((f, hh), lambda i: (0, 0)),
            pl.BlockSpec((hh,), lambda i: (0,)),
            pl.BlockSpec((f, 1), lambda i: (0, 0)),
            pl.BlockSpec((f, 1), lambda i: (0, 0)),
            pl.BlockSpec((1,), lambda i: (0,)),
            pl.BlockSpec((1,), lambda i: (0,)),
        ],
        out_specs=[
            pl.BlockSpec((_ROW_BLK, hh), lambda i: (i, 0)),
            pl.BlockSpec((_ROW_BLK, hh), lambda i: (i, 0)),
            pl.BlockSpec((_ROW_BLK, 1), lambda i: (i, 0)),
        ],
        out_shape=[
            jax.ShapeDtypeStruct((n, hh), jnp.float32),
            jax.ShapeDtypeStruct((n, hh), jnp.float32),
            jax.ShapeDtypeStruct((n, 1), jnp.float32),
        ],
    )(fea, W_in, W_in_self, b_in, scores0, Dk0, bias0, Dbias0)


def _tc_layer2(parts, W_out, W_out_self, b_out, scores0, Dk0, bias0, Dbias0):
    _, n, hh = parts.shape
    cc = W_out.shape[1]
    grid = (n // _ROW_BLK,)

    def body(p_ref, wo_ref, wos_ref, bo_ref, sc_ref, dk_ref, b0_ref,
             db_ref, s_out, d_out, sig_out):
        x = p_ref[0] + p_ref[1]
        S = jnp.dot(x, wo_ref[...], preferred_element_type=jnp.float32)
        sid = _sigmoid(
            jnp.dot(x, sc_ref[...], preferred_element_type=jnp.float32)
            + b0_ref[...])
        dk = jnp.dot(x, dk_ref[...], preferred_element_type=jnp.float32) \
            + db_ref[...]
        self_t = jnp.dot(x, wos_ref[...],
                         preferred_element_type=jnp.float32) + bo_ref[...]
        D = self_t + _GAMMA * dk * (S + self_t)
        s_out[...] = S
        d_out[...] = D
        sig_out[...] = sid

    return pl.pallas_call(
        body,
        grid=grid,
        in_specs=[
            pl.BlockSpec((2, _ROW_BLK, hh), lambda i: (0, i, 0)),
            pl.BlockSpec((hh, cc), lambda i: (0, 0)),
            pl.BlockSpec((hh, cc), lambda i: (0, 0)),
            pl.BlockSpec((cc,), lambda i: (0,)),
            pl.BlockSpec((hh, 1), lambda i: (0, 0)),
            pl.BlockSpec((hh, 1), lambda i: (0, 0)),
            pl.BlockSpec((1,), lambda i: (0,)),
            pl.BlockSpec((1,), lambda i: (0,)),
        ],
        out_specs=[
            pl.BlockSpec((_ROW_BLK, cc), lambda i: (i, 0)),
            pl.BlockSpec((_ROW_BLK, cc), lambda i: (i, 0)),
            pl.BlockSpec((_ROW_BLK, 1), lambda i: (i, 0)),
        ],
        out_shape=[
            jax.ShapeDtypeStruct((n, cc), jnp.float32),
            jax.ShapeDtypeStruct((n, cc), jnp.float32),
            jax.ShapeDtypeStruct((n, 1), jnp.float32),
        ],
    )(parts, W_out, W_out_self, b_out, scores0, Dk0, bias0, Dbias0)


def _tc_final(parts):
    _, n, cc = parts.shape
    grid = (n // _ROW_BLK,)

    def body(p_ref, o_ref):
        z = p_ref[0] + p_ref[1]
        m = jnp.max(z, axis=1, keepdims=True)
        zm = z - m
        o_ref[...] = zm - jnp.log(jnp.sum(jnp.exp(zm), axis=1, keepdims=True))

    return pl.pallas_call(
        body,
        grid=grid,
        in_specs=[pl.BlockSpec((2, _ROW_BLK, cc), lambda i: (0, i, 0))],
        out_specs=pl.BlockSpec((_ROW_BLK, cc), lambda i: (i, 0)),
        out_shape=jax.ShapeDtypeStruct((n, cc), jnp.float32),
    )(parts)


def _pack_edges(index, weight):
    e = weight.shape[0]
    unit = _NW * _B
    epad = ((e + unit - 1) // unit) * unit
    pad = epad - e
    src = jnp.concatenate([index[0], jnp.zeros((pad,), jnp.int32)])
    dst = jnp.concatenate([index[1], jnp.zeros((pad,), jnp.int32)])
    w = jnp.concatenate([weight, jnp.zeros((pad,), jnp.float32)])
    wi = lax.bitcast_convert_type(w, jnp.int32)
    nb = epad // _NW // _B
    comb = jnp.stack([x.reshape(_NW, nb, _B) for x in (src, dst, wi)],
                     axis=2)
    return comb, nb


def kernel(fea, adj_index, adj_weight, adj_knn_index, adj_knn_weight,
           W_in, W_in_self, b_in, W_out, W_out_self, b_out,
           scores0, bias0, Dk0, Dbias0):
    n_real = fea.shape[0]
    n = _N_PAD
    fea = jnp.pad(fea, ((0, n - n_real), (0, 0)))
    hh = W_in.shape[1]
    cc = W_out.shape[1]

    comb_adj, blocks_adj = _pack_edges(adj_index, adj_weight)
    comb_knn, blocks_knn = _pack_edges(adj_knn_index, adj_knn_weight)
    nb = blocks_adj + blocks_knn
    nb_pad = ((nb + _CH - 1) // _CH) * _CH
    comb = jnp.concatenate(
        [comb_adj, comb_knn] +
        ([jnp.zeros((_NW, nb_pad - nb, 3, _B), jnp.int32)]
         if nb_pad > nb else []), axis=1)
    nb = nb_pad

    S1, D1, sig1 = _tc_layer1(fea, W_in, W_in_self, b_in, scores0, Dk0,
                              bias0, Dbias0)
    sc1 = _make_sc_spmm(n, hh, blocks_adj, nb)
    parts1 = sc1(sig1.reshape(n), S1, comb, D1)

    S2, D2, sig2 = _tc_layer2(parts1, W_out, W_out_self, b_out, scores0,
                              Dk0, bias0, Dbias0)
    sc2 = _make_sc_spmm(n, cc, blocks_adj, nb)
    parts2 = sc2(sig2.reshape(n), S2, comb, D2)

    return _tc_final(parts2)[:n_real]


# speedup vs baseline: 6.4678x; 1.0023x over previous
import functools

import jax
import jax.numpy as jnp
from jax import lax
from jax.experimental import pallas as pl
from jax.experimental.pallas import tpu as pltpu
from jax.experimental.pallas import tpu_sc as plsc

_GAMMA = 0.1
_B = 64
_NW = 32
_ROW_BLK = 1024
_N_PAD = 10240


def _lane_bcast(v16, lane):
    idx = jnp.full((16, 1), lane, jnp.int32)
    return lax.gather(
        v16, idx,
        lax.GatherDimensionNumbers(
            offset_dims=(), collapsed_slice_dims=(0,), start_index_map=(0,)),
        slice_sizes=(1,),
        mode=lax.GatherScatterMode.PROMISE_IN_BOUNDS)


_CH = 4


def _make_sc_spmm(n, h, blocks_adj, nb):
    rpt = n // 16
    assert nb % _CH == 0
    nchunks = nb // _CH
    ngrp = _B // 16
    mesh = plsc.VectorSubcoreMesh(
        core_axis_name="c", subcore_axis_name="s", num_cores=2,
        num_subcores=16)

    @functools.partial(
        pl.kernel,
        out_type=jax.ShapeDtypeStruct((2, n, h), jnp.float32),
        mesh=mesh,
        scratch_types=[
            pltpu.VMEM((n,), jnp.float32),
            pltpu.VMEM((2, _CH, 3, _B), jnp.int32),
            pltpu.VMEM((_B, h), jnp.float32),
            pltpu.VMEM((_B, h), jnp.float32),
            pltpu.VMEM((_B, h), jnp.float32),
            pltpu.VMEM((_B, h), jnp.float32),
            pltpu.VMEM((8, h), jnp.float32),
            pltpu.VMEM_SHARED((n, h), jnp.float32),
            pltpu.SemaphoreType.DMA,
            pltpu.SemaphoreType.DMA,
            pltpu.SemaphoreType.DMA,
            pltpu.SemaphoreType.DMA,
            pltpu.SemaphoreType.DMA,
            pltpu.SemaphoreType.DMA,
            pltpu.SemaphoreType.DMA,
            pltpu.SemaphoreType.DMA,
            pltpu.SemaphoreType.DMA,
        ],
        compiler_params=pltpu.CompilerParams(
            needs_layout_passes=False, use_tc_tiling_on_sc=False),
    )
    def spmm_kernel(s_hbm, tab_hbm, comb_hbm, out_hbm,
                    s_v, comb_v, rows0, rows1, rows2, rows3, z_v, acc,
                    semg0, semg1, semg2, semg3,
                    sems0, sems1, sems2, sems3, semc):
        c = lax.axis_index("c")
        s = lax.axis_index("s")
        wid = c * 16 + s
        r0 = s * rpt

        zero = jnp.zeros((16,), jnp.float32)
        for r in range(8):
            for k in range(h // 16):
                z_v[r, pl.ds(k * 16, 16)] = zero

        def zblk(j, carry):
            pltpu.sync_copy(z_v, acc.at[pl.ds(r0 + j * 8, 8)])
            return carry

        lax.fori_loop(0, rpt // 8, zblk, 0)

        pltpu.sync_copy(s_hbm, s_v)
        pltpu.sync_copy(comb_hbm.at[wid, pl.ds(0, _CH)], comb_v.at[0])
        plsc.subcore_barrier()

        def stage_start(q):
            pltpu.async_copy(comb_hbm.at[wid, pl.ds(q * _CH, _CH)],
                             comb_v.at[q % 2], semc)

        def stage_wait(q):
            pltpu.make_async_copy(comb_hbm.at[wid, pl.ds(q * _CH, _CH)],
                                  comb_v.at[q % 2], semc).wait()

        def gather_start(qp, b, rows, semg):
            pltpu.async_copy(tab_hbm.at[comb_v.at[qp, b, 0]], rows, semg)

        def gather_wait(qp, b, rows, semg):
            pltpu.make_async_copy(tab_hbm.at[comb_v.at[qp, b, 0]], rows,
                                  semg).wait()

        def scatter_start(qp, b, rows, sems):
            pltpu.async_copy(rows, acc.at[comb_v.at[qp, b, 1]], sems,
                             add=True)

        def scatter_wait(qp, b, rows, sems):
            pltpu.make_async_copy(rows, acc.at[comb_v.at[qp, b, 1]],
                                  sems).wait()

        def scale(i, qp, b, rows):
            def grp(g, carry):
                gs = pl.ds(g * 16, 16)
                dst16 = comb_v[qp, b, 1, gs]
                w16 = plsc.bitcast(comb_v[qp, b, 2, gs], jnp.float32)
                sg = plsc.load_gather(s_v, [dst16])
                gate = jnp.where(i >= blocks_adj, 1.0 - sg, sg)
                ws16 = w16 * gate
                for lane in range(16):
                    wb = _lane_bcast(ws16, lane)
                    e = g * 16 + lane
                    for k in range(h // 16):
                        cs = pl.ds(k * 16, 16)
                        rows[e, cs] = rows[e, cs] * wb
                return carry

            lax.fori_loop(0, ngrp, grp, 0)

        bufs = [(rows0, semg0, sems0), (rows1, semg1, sems1),
                (rows2, semg2, sems2), (rows3, semg3, sems3)]
        gather_start(0, 0, rows0, semg0)

        def body(q, carry):
            qp = q % 2

            for b in range(_CH):
                i = q * _CH + b
                rows, semg, sems = bufs[b]
                rn, semg_n, sems_n = bufs[(b + 1) % _CH]

                if b == _CH - 1:
                    scatter_wait(qp, 0, rn, sems_n)
                else:
                    @pl.when(q >= 1)
                    def _():
                        scatter_wait(1 - qp, b + 1, rn, sems_n)

                if b == 2:
                    @pl.when(q + 1 < nchunks)
                    def _():
                        stage_start(q + 1)

                if b < _CH - 1:
                    gather_start(qp, b + 1, rn, semg_n)
                else:
                    @pl.when(q + 1 < nchunks)
                    def _():
                        stage_wait(q + 1)
                        gather_start(1 - qp, 0, rn, semg_n)

                gather_wait(qp, b, rows, semg)
                scale(i, qp, b, rows)
                scatter_start(qp, b, rows, sems)
            return carry

        lax.fori_loop(0, nchunks, body, 0)
        lq = (nchunks - 1) % 2
        for b in range(1, _CH):
            rows_l, _, sems_l = bufs[b]
            scatter_wait(lq, b, rows_l, sems_l)
        plsc.subcore_barrier()
        pltpu.sync_copy(acc.at[pl.ds(r0, rpt)],
                        out_hbm.at[c, pl.ds(r0, rpt)])

    return spmm_kernel


def _sigmoid(z):
    return 1.0 / (1.0 + jnp.exp(-z))


def _tc_layer1(fea, W_in, W_in_self, b_in, scores0, Dk0, bias0, Dbias0):
    n, f = fea.shape
    hh = W_in.shape[1]
    grid = (n // _ROW_BLK,)

    def body(f_ref, win_ref, wins_ref, bin_ref, sc_ref, dk_ref, b0_ref,
             db_ref, s_out, d_out, sig_out):
        x = f_ref[...]
        S = jnp.dot(x, win_ref[...], preferred_element_type=jnp.float32)
        sid = _sigmoid(
            jnp.dot(x, sc_ref[...], preferred_element_type=jnp.float32)
            + b0_ref[...])
        dk = jnp.dot(x, dk_ref[...], preferred_element_type=jnp.float32) \
            + db_ref[...]
        self_t = jnp.dot(x, wins_ref[...],
                         preferred_element_type=jnp.float32) + bin_ref[...]
        D = self_t + _GAMMA * dk * (S + self_t)
        s_out[...] = S
        d_out[...] = D
        sig_out[...] = sid

    return pl.pallas_call(
        body,
        grid=grid,
        in_specs=[
            pl.BlockSpec((_ROW_BLK, f), lambda i: (i, 0)),
            pl.BlockSpec((f, hh), lambda i: (0, 0)),
            pl.BlockSpec((f, hh), lambda i: (0, 0)),
            pl.BlockSpec((hh,), lambda i: (0,)),
            pl.BlockSpec((f, 1), lambda i: (0, 0)),
            pl.BlockSpec((f, 1), lambda i: (0, 0)),
            pl.BlockSpec((1,), lambda i: (0,)),
            pl.BlockSpec((1,), lambda i: (0,)),
        ],
        out_specs=[
            pl.BlockSpec((_ROW_BLK, hh), lambda i: (i, 0)),
            pl.BlockSpec((_ROW_BLK, hh), lambda i: (i, 0)),
            pl.BlockSpec((_ROW_BLK, 1), lambda i: (i, 0)),
        ],
        out_shape=[
            jax.ShapeDtypeStruct((n, hh), jnp.float32),
            jax.ShapeDtypeStruct((n, hh), jnp.float32),
            jax.ShapeDtypeStruct((n, 1), jnp.float32),
        ],
    )(fea, W_in, W_in_self, b_in, scores0, Dk0, bias0, Dbias0)


def _tc_layer2(parts, dense, W_out, W_out_self, b_out, scores0, Dk0, bias0,
               Dbias0):
    _, n, hh = parts.shape
    cc = W_out.shape[1]
    grid = (n // _ROW_BLK,)

    def body(p_ref, d_ref, wo_ref, wos_ref, bo_ref, sc_ref, dk_ref, b0_ref,
             db_ref, s_out, d_out, sig_out):
        x = p_ref[0] + p_ref[1] + d_ref[...]
        S = jnp.dot(x, wo_ref[...], preferred_element_type=jnp.float32)
        sid = _sigmoid(
            jnp.dot(x, sc_ref[...], preferred_element_type=jnp.float32)
            + b0_ref[...])
        dk = jnp.dot(x, dk_ref[...], preferred_element_type=jnp.float32) \
            + db_ref[...]
        self_t = jnp.dot(x, wos_ref[...],
                         preferred_element_type=jnp.float32) + bo_ref[...]
        D = self_t + _GAMMA * dk * (S + self_t)
        s_out[...] = S
        d_out[...] = D
        sig_out[...] = sid

    return pl.pallas_call(
        body,
        grid=grid,
        in_specs=[
            pl.BlockSpec((2, _ROW_BLK, hh), lambda i: (0, i, 0)),
            pl.BlockSpec((_ROW_BLK, hh), lambda i: (i, 0)),
            pl.BlockSpec((hh, cc), lambda i: (0, 0)),
            pl.BlockSpec((hh, cc), lambda i: (0, 0)),
            pl.BlockSpec((cc,), lambda i: (0,)),
            pl.BlockSpec((hh, 1), lambda i: (0, 0)),
            pl.BlockSpec((hh, 1), lambda i: (0, 0)),
            pl.BlockSpec((1,), lambda i: (0,)),
            pl.BlockSpec((1,), lambda i: (0,)),
        ],
        out_specs=[
            pl.BlockSpec((_ROW_BLK, cc), lambda i: (i, 0)),
            pl.BlockSpec((_ROW_BLK, cc), lambda i: (i, 0)),
            pl.BlockSpec((_ROW_BLK, 1), lambda i: (i, 0)),
        ],
        out_shape=[
            jax.ShapeDtypeStruct((n, cc), jnp.float32),
            jax.ShapeDtypeStruct((n, cc), jnp.float32),
            jax.ShapeDtypeStruct((n, 1), jnp.float32),
        ],
    )(parts, dense, W_out, W_out_self, b_out, scores0, Dk0, bias0, Dbias0)


def _tc_final(parts, dense):
    _, n, cc = parts.shape
    grid = (n // _ROW_BLK,)

    def body(p_ref, d_ref, o_ref):
        z = p_ref[0] + p_ref[1] + d_ref[...]
        m = jnp.max(z, axis=1, keepdims=True)
        zm = z - m
        o_ref[...] = zm - jnp.log(jnp.sum(jnp.exp(zm), axis=1, keepdims=True))

    return pl.pallas_call(
        body,
        grid=grid,
        in_specs=[pl.BlockSpec((2, _ROW_BLK, cc), lambda i: (0, i, 0)),
                  pl.BlockSpec((_ROW_BLK, cc), lambda i: (i, 0))],
        out_specs=pl.BlockSpec((_ROW_BLK, cc), lambda i: (i, 0)),
        out_shape=jax.ShapeDtypeStruct((n, cc), jnp.float32),
    )(parts, dense)


def _pack_edges(index, weight):
    e = weight.shape[0]
    unit = _NW * _B
    epad = ((e + unit - 1) // unit) * unit
    pad = epad - e
    src = jnp.concatenate([index[0], jnp.zeros((pad,), jnp.int32)])
    dst = jnp.concatenate([index[1], jnp.zeros((pad,), jnp.int32)])
    w = jnp.concatenate([weight, jnp.zeros((pad,), jnp.float32)])
    wi = lax.bitcast_convert_type(w, jnp.int32)
    nb = epad // _NW // _B
    comb = jnp.stack([x.reshape(_NW, nb, _B) for x in (src, dst, wi)],
                     axis=2)
    return comb, nb


def kernel(fea, adj_index, adj_weight, adj_knn_index, adj_knn_weight,
           W_in, W_in_self, b_in, W_out, W_out_self, b_out,
           scores0, bias0, Dk0, Dbias0):
    n_real = fea.shape[0]
    n = _N_PAD
    fea = jnp.pad(fea, ((0, n - n_real), (0, 0)))
    hh = W_in.shape[1]
    cc = W_out.shape[1]

    comb_adj, blocks_adj = _pack_edges(adj_index, adj_weight)
    comb_knn, blocks_knn = _pack_edges(adj_knn_index, adj_knn_weight)
    nb = blocks_adj + blocks_knn
    nb_pad = ((nb + _CH - 1) // _CH) * _CH
    comb = jnp.concatenate(
        [comb_adj, comb_knn] +
        ([jnp.zeros((_NW, nb_pad - nb, 3, _B), jnp.int32)]
         if nb_pad > nb else []), axis=1)
    nb = nb_pad

    S1, D1, sig1 = _tc_layer1(fea, W_in, W_in_self, b_in, scores0, Dk0,
                              bias0, Dbias0)
    sc1 = _make_sc_spmm(n, hh, blocks_adj, nb)
    parts1 = sc1(sig1.reshape(n), S1, comb)

    S2, D2, sig2 = _tc_layer2(parts1, D1, W_out, W_out_self, b_out, scores0,
                              Dk0, bias0, Dbias0)
    sc2 = _make_sc_spmm(n, cc, blocks_adj, nb)
    parts2 = sc2(sig2.reshape(n), S2, comb)

    return _tc_final(parts2, D2)[:n_real]
